# baseline (speedup 1.0000x reference)
import functools

import jax
import jax.numpy as jnp
from jax import lax
from jax.experimental import pallas as pl
from jax.experimental.pallas import tpu as pltpu
from jax.experimental.pallas import tpu_sc as plsc

N_ROWS = 320000
D = 128
N_SEG = 10000
EPS = 1e-05

L = 16
C = D // L
NC = 2
NS = 16
NW = NC * NS
SEG_PER_W = (N_SEG + NW - 1) // NW
B = 128
N_BOUNDS = 48


def _body(data_hbm, index_hbm, bounds_hbm, out_hbm,
          row_v, idx_v, segmin_v, segsum_v, bounds_v, sem):
    w = lax.axis_index("s") * NC + lax.axis_index("c")
    pltpu.sync_copy(bounds_hbm, bounds_v)
    bv = bounds_v[pl.ds(w, L)]
    row_lo = bv[0]
    row_hi = bv[1]
    seg_base = w * SEG_PER_W

    blk_lo = row_lo // B
    blk_hi = (row_hi + B - 1) // B

    inf_vec = jnp.full((L,), jnp.inf, jnp.float32)
    zero_vec = jnp.zeros((L,), jnp.float32)

    def init_body(i, _):
        for c in range(C):
            segmin_v[i, pl.ds(c * L, L)] = inf_vec
            segsum_v[i, pl.ds(c * L, L)] = zero_vec
        return 0
    lax.fori_loop(0, SEG_PER_W, init_body, 0)

    def block_bounds(b):
        r0 = b * B
        lo_i = jnp.maximum(row_lo - r0, 0)
        hi_i = jnp.minimum(row_hi - r0, B)
        return r0, lo_i, jnp.maximum(hi_i, lo_i)

    def p1_block(b, _):
        r0, lo_i, hi_i = block_bounds(b)
        pltpu.sync_copy(data_hbm.at[pl.ds(r0, B)], row_v)
        pltpu.sync_copy(index_hbm.at[pl.ds(r0, B)], idx_v.at[pl.ds(0, B)])

        def p1_row(i, _):
            s = idx_v[pl.ds(i, L)][0] - seg_base
            for c in range(C):
                v = row_v[i, pl.ds(c * L, L)]
                m = segmin_v[s, pl.ds(c * L, L)]
                segmin_v[s, pl.ds(c * L, L)] = jnp.minimum(m, v)
            return 0
        lax.fori_loop(lo_i, hi_i, p1_row, 0)
        return 0
    lax.fori_loop(blk_lo, blk_hi, p1_block, 0)

    def p2_block(b, _):
        r0, lo_i, hi_i = block_bounds(b)
        pltpu.sync_copy(data_hbm.at[pl.ds(r0, B)], row_v)
        pltpu.sync_copy(index_hbm.at[pl.ds(r0, B)], idx_v.at[pl.ds(0, B)])

        def p2_row(i, _):
            s = idx_v[pl.ds(i, L)][0] - seg_base
            for c in range(C):
                v = row_v[i, pl.ds(c * L, L)]
                m = segmin_v[s, pl.ds(c * L, L)]
                e = jnp.exp(v - m) + EPS
                acc = segsum_v[s, pl.ds(c * L, L)]
                segsum_v[s, pl.ds(c * L, L)] = acc + e
            return 0
        lax.fori_loop(lo_i, hi_i, p2_row, 0)
        return 0
    lax.fori_loop(blk_lo, blk_hi, p2_block, 0)

    def rcp_body(i, _):
        for c in range(C):
            segsum_v[i, pl.ds(c * L, L)] = 1.0 / segsum_v[i, pl.ds(c * L, L)]
        return 0
    lax.fori_loop(0, SEG_PER_W, rcp_body, 0)

    def p3_block(b, _):
        r0, lo_i, hi_i = block_bounds(b)
        pltpu.sync_copy(data_hbm.at[pl.ds(r0, B)], row_v)
        pltpu.sync_copy(index_hbm.at[pl.ds(r0, B)], idx_v.at[pl.ds(0, B)])

        def p3_row(i, _):
            s = idx_v[pl.ds(i, L)][0] - seg_base
            for c in range(C):
                v = row_v[i, pl.ds(c * L, L)]
                m = segmin_v[s, pl.ds(c * L, L)]
                e = jnp.exp(v - m) + EPS
                row_v[i, pl.ds(c * L, L)] = e * segsum_v[s, pl.ds(c * L, L)]
            return 0
        lax.fori_loop(lo_i, hi_i, p3_row, 0)

        full = jnp.logical_and(r0 >= row_lo, r0 + B <= row_hi)

        @pl.when(full)
        def _():
            pltpu.sync_copy(row_v, out_hbm.at[pl.ds(r0, B)])

        @pl.when(jnp.logical_not(full))
        def _():
            def tail_row(i, _):
                pltpu.sync_copy(row_v.at[pl.ds(i, 1)],
                                out_hbm.at[pl.ds(r0 + i, 1)])
                return 0
            lax.fori_loop(lo_i, hi_i, tail_row, 0)
        return 0
    lax.fori_loop(blk_lo, blk_hi, p3_block, 0)


@jax.jit
def _run(data, index_i32, bounds):
    mesh = plsc.VectorSubcoreMesh(core_axis_name="c", subcore_axis_name="s")
    f = pl.kernel(
        _body,
        mesh=mesh,
        out_type=jax.ShapeDtypeStruct((N_ROWS, D), jnp.float32),
        scratch_types=[
            pltpu.VMEM((B, D), jnp.float32),
            pltpu.VMEM((B + L,), jnp.int32),
            pltpu.VMEM((SEG_PER_W, D), jnp.float32),
            pltpu.VMEM((SEG_PER_W, D), jnp.float32),
            pltpu.VMEM((N_BOUNDS,), jnp.int32),
            pltpu.SemaphoreType.DMA,
        ],
    )
    return f(data, index_i32, bounds)


def kernel(data, index):
    index = index.astype(jnp.int32)
    targets = jnp.arange(NW + 1, dtype=jnp.int32) * SEG_PER_W
    bounds = jnp.searchsorted(index, targets, side="left").astype(jnp.int32)
    bounds = jnp.concatenate(
        [bounds, jnp.zeros((N_BOUNDS - NW - 1,), jnp.int32)])
    return _run(data, index, bounds)


# speedup vs baseline: 3.5802x; 3.5802x over previous
import jax
import jax.numpy as jnp
from jax import lax
from jax.experimental import pallas as pl
from jax.experimental.pallas import tpu as pltpu
from jax.experimental.pallas import tpu_sc as plsc

N_ROWS = 320000
D = 128
N_SEG = 10000
EPS = 1e-05

L = 16
C = D // L
NC = 2
NS = 16
NW = NC * NS
SEG_PER_W = (N_SEG + NW - 1) // NW
B = 128
N_BOUNDS = 48
MAX_RUNS = SEG_PER_W + 1


def _body(data_hbm, index_hbm, bounds_hbm, out_hbm,
          row_v, idx_v, runmin_v, runrcp_v, rstart_v, bounds_v, sem):
    w = lax.axis_index("s") * NC + lax.axis_index("c")
    pltpu.sync_copy(bounds_hbm, bounds_v)
    bv = bounds_v[pl.ds(w, L)]
    row_lo = bv[0]
    row_hi = bv[1]

    blk_lo = row_lo // B
    blk_hi = (row_hi + B - 1) // B

    zero_vec = jnp.zeros((L,), jnp.float32)
    zeros = (zero_vec,) * C

    def load_block(b, with_idx):
        r0 = b * B
        pltpu.sync_copy(data_hbm.at[pl.ds(r0, B)], row_v)
        if with_idx:
            pltpu.sync_copy(index_hbm.at[pl.ds(r0, B)], idx_v.at[pl.ds(0, B)])

    def load_row(i):
        return tuple(row_v[i, pl.ds(c * L, L)] for c in range(C))

    def read_rstart(j):
        return rstart_v[pl.ds(j * L, L)][0]

    def p1_block(b, carry):
        r0 = b * B
        lo_i = jnp.maximum(row_lo - r0, 0)
        hi_i = jnp.maximum(jnp.minimum(row_hi - r0, B), lo_i)
        load_block(b, True)

        def row_fn(i, c_):
            n, cur_s = c_[0], c_[1]
            m = c_[2:]
            s = idx_v[pl.ds(i, L)][0]
            changed = s != cur_s
            n2 = jnp.where(changed, n + 1, n)
            dst_s = jnp.where(changed, n2, 0)
            rstart_v[pl.ds(dst_s * L, L)] = \
                jnp.zeros((L,), jnp.int32) + (r0 + i)
            v = load_row(i)
            m2 = tuple(
                jnp.where(changed, v[c], jnp.minimum(m[c], v[c]))
                for c in range(C))
            for c in range(C):
                runmin_v[n2, pl.ds(c * L, L)] = m2[c]
            return (n2, s) + m2

        return lax.fori_loop(lo_i, hi_i, row_fn, carry)

    fin = lax.fori_loop(blk_lo, blk_hi, p1_block,
                        (jnp.int32(0), jnp.int32(-1)) + zeros)
    n_runs = fin[0]
    rstart_v[pl.ds((n_runs + 1) * L, L)] = jnp.zeros((L,), jnp.int32) + row_hi

    def p2_run(j, cur_blk):
        start = read_rstart(j)
        end = read_rstart(j + 1)
        m = tuple(runmin_v[j, pl.ds(c * L, L)] for c in range(C))
        b0 = start // B
        b1 = (end + B - 1) // B

        def piece(b, carry):
            prev_blk = carry[0]
            a = carry[1:]

            @pl.when(b != prev_blk)
            def _():
                load_block(b, False)

            r0 = b * B
            lo_i = jnp.maximum(start - r0, 0)
            hi_i = jnp.minimum(end - r0, B)

            def row_fn(i, a_):
                v = load_row(i)
                return tuple(
                    a_[c] + (jnp.exp(v[c] - m[c]) + EPS) for c in range(C))

            a2 = lax.fori_loop(lo_i, hi_i, row_fn, a)
            return (b,) + a2

        fin2 = lax.fori_loop(b0, b1, piece, (cur_blk,) + zeros)
        a = fin2[1:]
        for c in range(C):
            runrcp_v[j, pl.ds(c * L, L)] = 1.0 / a[c]
        return fin2[0]

    lax.fori_loop(1, n_runs + 1, p2_run, jnp.int32(-1))

    def writeback(blk):
        r0 = blk * B
        lo_i = jnp.maximum(row_lo - r0, 0)
        hi_i = jnp.maximum(jnp.minimum(row_hi - r0, B), lo_i)
        full = jnp.logical_and(lo_i == 0, hi_i == B)

        @pl.when(full)
        def _():
            pltpu.sync_copy(row_v, out_hbm.at[pl.ds(r0, B)])

        @pl.when(jnp.logical_not(full))
        def _():
            def tail_row(i, _):
                pltpu.sync_copy(row_v.at[pl.ds(i, 1)],
                                out_hbm.at[pl.ds(r0 + i, 1)])
                return 0
            lax.fori_loop(lo_i, hi_i, tail_row, 0)

    def p3_run(j, cur_blk):
        start = read_rstart(j)
        end = read_rstart(j + 1)
        m = tuple(runmin_v[j, pl.ds(c * L, L)] for c in range(C))
        rc = tuple(runrcp_v[j, pl.ds(c * L, L)] for c in range(C))
        b0 = start // B
        b1 = (end + B - 1) // B

        def piece(b, prev_blk):
            @pl.when(jnp.logical_and(b != prev_blk, prev_blk >= 0))
            def _():
                writeback(prev_blk)

            @pl.when(b != prev_blk)
            def _():
                load_block(b, False)

            r0 = b * B
            lo_i = jnp.maximum(start - r0, 0)
            hi_i = jnp.minimum(end - r0, B)

            def row_fn(i, z):
                v = load_row(i)
                for c in range(C):
                    row_v[i, pl.ds(c * L, L)] = \
                        (jnp.exp(v[c] - m[c]) + EPS) * rc[c]
                return z

            lax.fori_loop(lo_i, hi_i, row_fn, 0)
            return b

        return lax.fori_loop(b0, b1, piece, cur_blk)

    last_blk = lax.fori_loop(1, n_runs + 1, p3_run, jnp.int32(-1))

    @pl.when(last_blk >= 0)
    def _():
        writeback(last_blk)


@jax.jit
def _run(data, index_i32, bounds):
    mesh = plsc.VectorSubcoreMesh(core_axis_name="c", subcore_axis_name="s")
    f = pl.kernel(
        _body,
        mesh=mesh,
        out_type=jax.ShapeDtypeStruct((N_ROWS, D), jnp.float32),
        scratch_types=[
            pltpu.VMEM((B, D), jnp.float32),
            pltpu.VMEM((B + L,), jnp.int32),
            pltpu.VMEM((MAX_RUNS + 1, D), jnp.float32),
            pltpu.VMEM((MAX_RUNS + 1, D), jnp.float32),
            pltpu.VMEM(((MAX_RUNS + 2) * L,), jnp.int32),
            pltpu.VMEM((N_BOUNDS,), jnp.int32),
            pltpu.SemaphoreType.DMA,
        ],
    )
    return f(data, index_i32, bounds)


def kernel(data, index):
    index = index.astype(jnp.int32)
    targets = jnp.arange(NW + 1, dtype=jnp.int32) * SEG_PER_W
    bounds = jnp.searchsorted(index, targets, side="left").astype(jnp.int32)
    bounds = jnp.concatenate(
        [bounds, jnp.zeros((N_BOUNDS - NW - 1,), jnp.int32)])
    return _run(data, index, bounds)


# speedup vs baseline: 5.4438x; 1.5205x over previous
import jax
import jax.numpy as jnp
from jax import lax
from jax.experimental import pallas as pl
from jax.experimental.pallas import tpu as pltpu
from jax.experimental.pallas import tpu_sc as plsc

N_ROWS = 320000
D = 128
N_SEG = 10000
EPS = 1e-05

L = 16
C = D // L
NC = 2
NS = 16
NW = NC * NS
SEG_PER_W = (N_SEG + NW - 1) // NW
B = 128
N_BOUNDS = 48
MAX_RUNS = SEG_PER_W + 1


def _body(data_hbm, index_hbm, bounds_hbm, out_hbm,
          row_v, idx_v, runmin_v, runrcp_v, rstart_v, bounds_v,
          sem_d, sem_i):
    w = lax.axis_index("s") * NC + lax.axis_index("c")
    pltpu.sync_copy(bounds_hbm, bounds_v)
    bv = bounds_v[pl.ds(w, L)]
    row_lo = bv[0]
    row_hi = bv[1]

    blk_lo = row_lo // B
    blk_hi = (row_hi + B - 1) // B

    zero_vec = jnp.zeros((L,), jnp.float32)
    zeros = (zero_vec,) * C

    def in_copies(b, with_idx):
        p = lax.rem(b, 2)
        r0 = b * B
        cps = [pltpu.make_async_copy(
            data_hbm.at[pl.ds(r0, B)], row_v.at[p], sem_d.at[p])]
        if with_idx:
            cps.append(pltpu.make_async_copy(
                index_hbm.at[pl.ds(r0, B)], idx_v.at[pl.ds(p * (B + L), B)],
                sem_i.at[p]))
        return cps

    def start_in(b, with_idx):
        for cp in in_copies(b, with_idx):
            cp.start()

    def wait_in(b, with_idx):
        for cp in in_copies(b, with_idx):
            cp.wait()

    def stage(b, prev_blk, with_idx):
        @pl.when(b != prev_blk)
        def _():
            wait_in(b, with_idx)

        @pl.when(jnp.logical_and(b != prev_blk, b + 1 < blk_hi))
        def _():
            start_in(b + 1, with_idx)

    def load_row(p, i):
        return tuple(row_v[p, i, pl.ds(c * L, L)] for c in range(C))

    def read_rstart(j):
        return rstart_v[pl.ds(j * L, L)][0]

    def p1_block(b, carry):
        r0 = b * B
        p = lax.rem(b, 2)
        lo_i = jnp.maximum(row_lo - r0, 0)
        hi_i = jnp.maximum(jnp.minimum(row_hi - r0, B), lo_i)
        stage(b, b - 1, True)

        def row_fn(i, c_):
            n, cur_s = c_[0], c_[1]
            m = c_[2:]
            s = idx_v[pl.ds(p * (B + L) + i, L)][0]
            changed = s != cur_s
            n2 = jnp.where(changed, n + 1, n)
            dst_s = jnp.where(changed, n2, 0)
            rstart_v[pl.ds(dst_s * L, L)] = \
                jnp.zeros((L,), jnp.int32) + (r0 + i)
            v = load_row(p, i)
            m2 = tuple(
                jnp.where(changed, v[c], jnp.minimum(m[c], v[c]))
                for c in range(C))
            for c in range(C):
                runmin_v[n2, pl.ds(c * L, L)] = m2[c]
            return (n2, s) + m2

        return lax.fori_loop(lo_i, hi_i, row_fn, carry)

    @pl.when(blk_lo < blk_hi)
    def _():
        start_in(blk_lo, True)

    fin = lax.fori_loop(blk_lo, blk_hi, p1_block,
                        (jnp.int32(0), jnp.int32(-1)) + zeros)
    n_runs = fin[0]
    rstart_v[pl.ds((n_runs + 1) * L, L)] = jnp.zeros((L,), jnp.int32) + row_hi

    def p2_run(j, cur_blk):
        start = read_rstart(j)
        end = read_rstart(j + 1)
        m = tuple(runmin_v[j, pl.ds(c * L, L)] for c in range(C))
        b0 = start // B
        b1 = (end + B - 1) // B

        def piece(b, carry):
            prev_blk = carry[0]
            a = carry[1:]
            stage(b, prev_blk, False)
            p = lax.rem(b, 2)

            r0 = b * B
            lo_i = jnp.maximum(start - r0, 0)
            hi_i = jnp.minimum(end - r0, B)

            def row_fn(i, a_):
                v = load_row(p, i)
                return tuple(
                    a_[c] + (jnp.exp(v[c] - m[c]) + EPS) for c in range(C))

            a2 = lax.fori_loop(lo_i, hi_i, row_fn, a)
            return (b,) + a2

        fin2 = lax.fori_loop(b0, b1, piece, (cur_blk,) + zeros)
        a = fin2[1:]
        for c in range(C):
            runrcp_v[j, pl.ds(c * L, L)] = 1.0 / a[c]
        return fin2[0]

    @pl.when(blk_lo < blk_hi)
    def _():
        start_in(blk_lo, False)

    lax.fori_loop(1, n_runs + 1, p2_run, jnp.int32(-1))

    def writeback(blk):
        r0 = blk * B
        lo_i = jnp.maximum(row_lo - r0, 0)
        hi_i = jnp.maximum(jnp.minimum(row_hi - r0, B), lo_i)
        full = jnp.logical_and(lo_i == 0, hi_i == B)

        p = lax.rem(blk, 2)

        @pl.when(full)
        def _():
            pltpu.sync_copy(row_v.at[p], out_hbm.at[pl.ds(r0, B)])

        @pl.when(jnp.logical_not(full))
        def _():
            def tail_row(i, _):
                pltpu.sync_copy(row_v.at[p, pl.ds(i, 1)],
                                out_hbm.at[pl.ds(r0 + i, 1)])
                return 0
            lax.fori_loop(lo_i, hi_i, tail_row, 0)

    def p3_run(j, cur_blk):
        start = read_rstart(j)
        end = read_rstart(j + 1)
        m = tuple(runmin_v[j, pl.ds(c * L, L)] for c in range(C))
        rc = tuple(runrcp_v[j, pl.ds(c * L, L)] for c in range(C))
        b0 = start // B
        b1 = (end + B - 1) // B

        def piece(b, prev_blk):
            @pl.when(jnp.logical_and(b != prev_blk, prev_blk >= 0))
            def _():
                writeback(prev_blk)

            stage(b, prev_blk, False)
            p = lax.rem(b, 2)

            r0 = b * B
            lo_i = jnp.maximum(start - r0, 0)
            hi_i = jnp.minimum(end - r0, B)

            def row_fn(i, z):
                v = load_row(p, i)
                for c in range(C):
                    row_v[p, i, pl.ds(c * L, L)] = \
                        (jnp.exp(v[c] - m[c]) + EPS) * rc[c]
                return z

            lax.fori_loop(lo_i, hi_i, row_fn, 0)
            return b

        return lax.fori_loop(b0, b1, piece, cur_blk)

    @pl.when(blk_lo < blk_hi)
    def _():
        start_in(blk_lo, False)

    last_blk = lax.fori_loop(1, n_runs + 1, p3_run, jnp.int32(-1))

    @pl.when(last_blk >= 0)
    def _():
        writeback(last_blk)


@jax.jit
def _run(data, index_i32, bounds):
    mesh = plsc.VectorSubcoreMesh(core_axis_name="c", subcore_axis_name="s")
    f = pl.kernel(
        _body,
        mesh=mesh,
        out_type=jax.ShapeDtypeStruct((N_ROWS, D), jnp.float32),
        scratch_types=[
            pltpu.VMEM((2, B, D), jnp.float32),
            pltpu.VMEM((2 * (B + L),), jnp.int32),
            pltpu.VMEM((MAX_RUNS + 1, D), jnp.float32),
            pltpu.VMEM((MAX_RUNS + 1, D), jnp.float32),
            pltpu.VMEM(((MAX_RUNS + 2) * L,), jnp.int32),
            pltpu.VMEM((N_BOUNDS,), jnp.int32),
            pltpu.SemaphoreType.DMA((2,)),
            pltpu.SemaphoreType.DMA((2,)),
        ],
    )
    return f(data, index_i32, bounds)


def kernel(data, index):
    index = index.astype(jnp.int32)
    targets = jnp.arange(NW + 1, dtype=jnp.int32) * SEG_PER_W
    bounds = jnp.searchsorted(index, targets, side="left").astype(jnp.int32)
    bounds = jnp.concatenate(
        [bounds, jnp.zeros((N_BOUNDS - NW - 1,), jnp.int32)])
    return _run(data, index, bounds)


# speedup vs baseline: 5.5853x; 1.0260x over previous
import jax
import jax.numpy as jnp
from jax import lax
from jax.experimental import pallas as pl
from jax.experimental.pallas import tpu as pltpu
from jax.experimental.pallas import tpu_sc as plsc

N_ROWS = 320000
D = 128
N_SEG = 10000
EPS = 1e-05

L = 16
C = D // L
NC = 2
NS = 16
NW = NC * NS
SEG_PER_W = (N_SEG + NW - 1) // NW
B = 80
N_BOUNDS = 48
MAX_RUNS = SEG_PER_W + 1


def _body(data_hbm, index_hbm, bounds_hbm, out_hbm,
          row_v, idx_v, runmin_v, runrcp_v, rstart_v, bounds_v,
          sem_d, sem_i, sem_o):
    w = lax.axis_index("s") * NC + lax.axis_index("c")
    pltpu.sync_copy(bounds_hbm, bounds_v)
    bv = bounds_v[pl.ds(w, L)]
    row_lo = bv[0]
    row_hi = bv[1]

    blk_lo = row_lo // B
    blk_hi = (row_hi + B - 1) // B

    zero_vec = jnp.zeros((L,), jnp.float32)
    zeros = (zero_vec,) * C

    def in_copies(b, with_idx):
        p = lax.rem(b, 3)
        r0 = b * B
        cps = [pltpu.make_async_copy(
            data_hbm.at[pl.ds(r0, B)], row_v.at[p], sem_d.at[p])]
        if with_idx:
            cps.append(pltpu.make_async_copy(
                index_hbm.at[pl.ds(r0, B)], idx_v.at[pl.ds(p * (B + L), B)],
                sem_i.at[p]))
        return cps

    def start_in(b, with_idx):
        for cp in in_copies(b, with_idx):
            cp.start()

    def wait_in(b, with_idx):
        for cp in in_copies(b, with_idx):
            cp.wait()

    def stage(b, prev_blk, with_idx):
        @pl.when(b != prev_blk)
        def _():
            wait_in(b, with_idx)

        @pl.when(jnp.logical_and(b != prev_blk, b + 1 < blk_hi))
        def _():
            start_in(b + 1, with_idx)

    def load_row(p, i):
        return tuple(row_v[p, i, pl.ds(c * L, L)] for c in range(C))

    def read_rstart(j):
        return rstart_v[pl.ds(j * L, L)][0]

    def p1_block(b, carry):
        r0 = b * B
        p = lax.rem(b, 3)
        lo_i = jnp.maximum(row_lo - r0, 0)
        hi_i = jnp.maximum(jnp.minimum(row_hi - r0, B), lo_i)
        stage(b, b - 1, True)

        def row_fn(i, c_):
            n, cur_s = c_[0], c_[1]
            m = c_[2:]
            s = idx_v[pl.ds(p * (B + L) + i, L)][0]
            changed = s != cur_s
            n2 = jnp.where(changed, n + 1, n)
            dst_s = jnp.where(changed, n2, 0)
            rstart_v[pl.ds(dst_s * L, L)] = \
                jnp.zeros((L,), jnp.int32) + (r0 + i)
            v = load_row(p, i)
            m2 = tuple(
                jnp.where(changed, v[c], jnp.minimum(m[c], v[c]))
                for c in range(C))
            for c in range(C):
                runmin_v[n2, pl.ds(c * L, L)] = m2[c]
            return (n2, s) + m2

        return lax.fori_loop(lo_i, hi_i, row_fn, carry)

    @pl.when(blk_lo < blk_hi)
    def _():
        start_in(blk_lo, True)

    fin = lax.fori_loop(blk_lo, blk_hi, p1_block,
                        (jnp.int32(0), jnp.int32(-1)) + zeros)
    n_runs = fin[0]
    rstart_v[pl.ds((n_runs + 1) * L, L)] = jnp.zeros((L,), jnp.int32) + row_hi

    def p2_run(j, cur_blk):
        start = read_rstart(j)
        end = read_rstart(j + 1)
        m = tuple(runmin_v[j, pl.ds(c * L, L)] for c in range(C))
        b0 = start // B
        b1 = (end + B - 1) // B

        def piece(b, carry):
            prev_blk = carry[0]
            a = carry[1:]
            stage(b, prev_blk, False)
            p = lax.rem(b, 3)

            r0 = b * B
            lo_i = jnp.maximum(start - r0, 0)
            hi_i = jnp.minimum(end - r0, B)

            def row_fn(i, a_):
                v = load_row(p, i)
                return tuple(
                    a_[c] + (jnp.exp(v[c] - m[c]) + EPS) for c in range(C))

            a2 = lax.fori_loop(lo_i, hi_i, row_fn, a)
            return (b,) + a2

        fin2 = lax.fori_loop(b0, b1, piece, (cur_blk,) + zeros)
        a = fin2[1:]
        for c in range(C):
            runrcp_v[j, pl.ds(c * L, L)] = 1.0 / a[c]
        return fin2[0]

    @pl.when(blk_lo < blk_hi)
    def _():
        start_in(blk_lo, False)

    lax.fori_loop(1, n_runs + 1, p2_run, jnp.int32(-1))

    def is_full(blk):
        r0 = blk * B
        return jnp.logical_and(r0 >= row_lo, r0 + B <= row_hi)

    def out_copy(blk):
        p = lax.rem(blk, 3)
        return pltpu.make_async_copy(
            row_v.at[p], out_hbm.at[pl.ds(blk * B, B)], sem_o.at[p])

    def wb_edge(blk):
        r0 = blk * B
        p = lax.rem(blk, 3)
        lo_i = jnp.maximum(row_lo - r0, 0)
        hi_i = jnp.maximum(jnp.minimum(row_hi - r0, B), lo_i)

        def tail_row(i, _):
            pltpu.sync_copy(row_v.at[p, pl.ds(i, 1)],
                            out_hbm.at[pl.ds(r0 + i, 1)])
            return 0
        lax.fori_loop(lo_i, hi_i, tail_row, 0)

    def p3_run(j, cur_blk):
        start = read_rstart(j)
        end = read_rstart(j + 1)
        m = tuple(runmin_v[j, pl.ds(c * L, L)] for c in range(C))
        rc = tuple(runrcp_v[j, pl.ds(c * L, L)] for c in range(C))
        b0 = start // B
        b1 = (end + B - 1) // B

        def piece(b, prev_blk):
            newb = b != prev_blk
            has_prev = jnp.logical_and(newb, prev_blk >= 0)

            @pl.when(jnp.logical_and(has_prev, is_full(prev_blk)))
            def _():
                out_copy(prev_blk).start()

            @pl.when(jnp.logical_and(has_prev,
                                     jnp.logical_not(is_full(prev_blk))))
            def _():
                wb_edge(prev_blk)

            @pl.when(newb)
            def _():
                wait_in(b, False)

            pref = jnp.logical_and(newb, b + 1 < blk_hi)

            @pl.when(jnp.logical_and(pref,
                                     jnp.logical_and(b - 2 >= blk_lo,
                                                     is_full(b - 2))))
            def _():
                out_copy(b - 2).wait()

            @pl.when(pref)
            def _():
                start_in(b + 1, False)

            p = lax.rem(b, 3)
            r0 = b * B
            lo_i = jnp.maximum(start - r0, 0)
            hi_i = jnp.minimum(end - r0, B)

            def row_fn(i, z):
                v = load_row(p, i)
                for c in range(C):
                    row_v[p, i, pl.ds(c * L, L)] = \
                        (jnp.exp(v[c] - m[c]) + EPS) * rc[c]
                return z

            lax.fori_loop(lo_i, hi_i, row_fn, 0)
            return b

        return lax.fori_loop(b0, b1, piece, cur_blk)

    @pl.when(blk_lo < blk_hi)
    def _():
        start_in(blk_lo, False)

    last_blk = lax.fori_loop(1, n_runs + 1, p3_run, jnp.int32(-1))

    def drain(blk):
        @pl.when(jnp.logical_and(last_blk >= 0,
                                 jnp.logical_and(blk >= blk_lo,
                                                 is_full(blk))))
        def _():
            out_copy(blk).wait()

    drain(last_blk - 2)
    drain(last_blk - 1)

    @pl.when(jnp.logical_and(last_blk >= 0, is_full(last_blk)))
    def _():
        p = lax.rem(last_blk, 3)
        pltpu.sync_copy(row_v.at[p], out_hbm.at[pl.ds(last_blk * B, B)])

    @pl.when(jnp.logical_and(last_blk >= 0,
                             jnp.logical_not(is_full(last_blk))))
    def _():
        wb_edge(last_blk)


@jax.jit
def _run(data, index_i32, bounds):
    mesh = plsc.VectorSubcoreMesh(core_axis_name="c", subcore_axis_name="s")
    f = pl.kernel(
        _body,
        mesh=mesh,
        out_type=jax.ShapeDtypeStruct((N_ROWS, D), jnp.float32),
        scratch_types=[
            pltpu.VMEM((3, B, D), jnp.float32),
            pltpu.VMEM((3 * (B + L),), jnp.int32),
            pltpu.VMEM((MAX_RUNS + 1, D), jnp.float32),
            pltpu.VMEM((MAX_RUNS + 1, D), jnp.float32),
            pltpu.VMEM(((MAX_RUNS + 2) * L,), jnp.int32),
            pltpu.VMEM((N_BOUNDS,), jnp.int32),
            pltpu.SemaphoreType.DMA((3,)),
            pltpu.SemaphoreType.DMA((3,)),
            pltpu.SemaphoreType.DMA((3,)),
        ],
    )
    return f(data, index_i32, bounds)


def kernel(data, index):
    index = index.astype(jnp.int32)
    targets = jnp.arange(NW + 1, dtype=jnp.int32) * SEG_PER_W
    bounds = jnp.searchsorted(index, targets, side="left").astype(jnp.int32)
    bounds = jnp.concatenate(
        [bounds, jnp.zeros((N_BOUNDS - NW - 1,), jnp.int32)])
    return _run(data, index, bounds)
